# Initial kernel scaffold; baseline (speedup 1.0000x reference)
import jax
import jax.numpy as jnp
from jax.experimental import pallas as pl


def kernel(x, weight):
    raise NotImplementedError("write your pallas kernel here")


# speedup vs baseline: 2.2711x; 2.2711x over previous
import jax
import jax.numpy as jnp
from jax.experimental import pallas as pl
from jax.experimental.pallas import tpu as pltpu

BM = 64
BK = 64
TOPK = 16
RT = 512
GB = RT // BM
TM = 512
TN = 1024


def _mask_kernel(x_ref, xm_ref):
    xv = x_ref[...]
    a3 = jnp.abs(xv).reshape(GB, BM, xv.shape[1])
    rs = jnp.sum(a3, axis=1)
    kb = xv.shape[1] // BK
    c_idx = jax.lax.broadcasted_iota(jnp.int32, (xv.shape[1], kb), 0)
    b_idx = jax.lax.broadcasted_iota(jnp.int32, (xv.shape[1], kb), 1)
    B = jnp.where((c_idx // BK) == b_idx, 1.0, 0.0)
    mag = jax.lax.dot(rs, B, precision=jax.lax.Precision.HIGHEST)
    mj = mag[:, :, None]
    mb = mag[:, None, :]
    jj = jax.lax.broadcasted_iota(jnp.int32, (GB, kb, kb), 1)
    bb = jax.lax.broadcasted_iota(jnp.int32, (GB, kb, kb), 2)
    beats = (mj > mb) | ((mj == mb) & (jj < bb))
    rank = jnp.sum(jnp.where(beats, 1.0, 0.0), axis=1)
    selb = jnp.where(rank < float(TOPK), 1.0, 0.0)
    r_idx = jax.lax.broadcasted_iota(jnp.int32, (kb, xv.shape[1]), 0)
    c2 = jax.lax.broadcasted_iota(jnp.int32, (kb, xv.shape[1]), 1)
    BT = jnp.where(r_idx == (c2 // BK), 1.0, 0.0)
    selL = jax.lax.dot(selb, BT)
    xm = xv.reshape(GB, BM, xv.shape[1]) * selL[:, None, :]
    xm_ref[...] = xm.reshape(xv.shape).astype(jnp.bfloat16)


def _mm_kernel(xm_ref, w_ref, o_ref):
    o_ref[...] = jnp.dot(xm_ref[...], w_ref[...],
                         preferred_element_type=jnp.float32)


def kernel(x, weight):
    m, h = x.shape
    n = weight.shape[1]

    xm = pl.pallas_call(
        _mask_kernel,
        out_shape=jax.ShapeDtypeStruct((m, h), jnp.bfloat16),
        grid=(m // RT,),
        in_specs=[pl.BlockSpec((RT, h), lambda i: (i, 0))],
        out_specs=pl.BlockSpec((RT, h), lambda i: (i, 0)),
        compiler_params=pltpu.CompilerParams(
            dimension_semantics=("parallel",),
            vmem_limit_bytes=40 * 1024 * 1024,
        ),
        name="mask_topk",
    )(x)

    wb = weight.astype(jnp.bfloat16)

    out = pl.pallas_call(
        _mm_kernel,
        out_shape=jax.ShapeDtypeStruct((m, n), jnp.float32),
        grid=(n // TN, m // TM),
        in_specs=[
            pl.BlockSpec((TM, h), lambda j, i: (i, 0)),
            pl.BlockSpec((h, TN), lambda j, i: (0, j)),
        ],
        out_specs=pl.BlockSpec((TM, TN), lambda j, i: (i, j)),
        compiler_params=pltpu.CompilerParams(
            dimension_semantics=("arbitrary", "arbitrary"),
            vmem_limit_bytes=48 * 1024 * 1024,
        ),
        name="masked_matmul",
    )(xm, wb)
    return out


# speedup vs baseline: 2.3066x; 1.0156x over previous
import jax
import jax.numpy as jnp
from jax.experimental import pallas as pl
from jax.experimental.pallas import tpu as pltpu

BM = 64
BK = 64
TOPK = 16
RT = 512
GB = RT // BM
TM = 512
TN = 2048


def _mask_kernel(x_ref, xm_ref):
    xv = x_ref[...]
    a3 = jnp.abs(xv).reshape(GB, BM, xv.shape[1])
    rs = jnp.sum(a3, axis=1)
    kb = xv.shape[1] // BK
    c_idx = jax.lax.broadcasted_iota(jnp.int32, (xv.shape[1], kb), 0)
    b_idx = jax.lax.broadcasted_iota(jnp.int32, (xv.shape[1], kb), 1)
    B = jnp.where((c_idx // BK) == b_idx, 1.0, 0.0)
    mag = jax.lax.dot(rs, B, precision=jax.lax.Precision.HIGHEST)
    mj = mag[:, :, None]
    mb = mag[:, None, :]
    jj = jax.lax.broadcasted_iota(jnp.int32, (GB, kb, kb), 1)
    bb = jax.lax.broadcasted_iota(jnp.int32, (GB, kb, kb), 2)
    beats = (mj > mb) | ((mj == mb) & (jj < bb))
    rank = jnp.sum(jnp.where(beats, 1.0, 0.0), axis=1)
    selb = jnp.where(rank < float(TOPK), 1.0, 0.0)
    r_idx = jax.lax.broadcasted_iota(jnp.int32, (kb, xv.shape[1]), 0)
    c2 = jax.lax.broadcasted_iota(jnp.int32, (kb, xv.shape[1]), 1)
    BT = jnp.where(r_idx == (c2 // BK), 1.0, 0.0)
    selL = jax.lax.dot(selb, BT)
    xm = xv.reshape(GB, BM, xv.shape[1]) * selL[:, None, :]
    xm_ref[...] = xm.reshape(xv.shape).astype(jnp.bfloat16)


def _mm_kernel(xm_ref, w_ref, o_ref):
    o_ref[...] = jnp.dot(xm_ref[...], w_ref[...],
                         preferred_element_type=jnp.float32)


def kernel(x, weight):
    m, h = x.shape
    n = weight.shape[1]

    xm = pl.pallas_call(
        _mask_kernel,
        out_shape=jax.ShapeDtypeStruct((m, h), jnp.bfloat16),
        grid=(m // RT,),
        in_specs=[pl.BlockSpec((RT, h), lambda i: (i, 0))],
        out_specs=pl.BlockSpec((RT, h), lambda i: (i, 0)),
        compiler_params=pltpu.CompilerParams(
            dimension_semantics=("parallel",),
            vmem_limit_bytes=40 * 1024 * 1024,
        ),
        name="mask_topk",
    )(x)

    wb = weight.astype(jnp.bfloat16)

    out = pl.pallas_call(
        _mm_kernel,
        out_shape=jax.ShapeDtypeStruct((m, n), jnp.float32),
        grid=(n // TN, m // TM),
        in_specs=[
            pl.BlockSpec((TM, h), lambda j, i: (i, 0)),
            pl.BlockSpec((h, TN), lambda j, i: (0, j)),
        ],
        out_specs=pl.BlockSpec((TM, TN), lambda j, i: (i, j)),
        compiler_params=pltpu.CompilerParams(
            dimension_semantics=("arbitrary", "arbitrary"),
            vmem_limit_bytes=56 * 1024 * 1024,
        ),
        name="masked_matmul",
    )(xm, wb)
    return out
